# Initial kernel scaffold; baseline (speedup 1.0000x reference)
import jax
import jax.numpy as jnp
from jax.experimental import pallas as pl


def kernel(users, x, edge_index, W1, b1, W2, b2):
    raise NotImplementedError("write your pallas kernel here")


# speedup vs baseline: 13.2635x; 13.2635x over previous
import functools

import jax
import jax.numpy as jnp
from jax import lax
from jax.experimental import pallas as pl
from jax.experimental.pallas import tpu as pltpu
from jax.experimental.pallas import tpu_sc as plsc

NC = 2
NS = 16
NW = NC * NS
L = 16

N = 10000
E = 320000
D = 128
B = 4096

EK = 80
E_PER = E // NW
ECH = E_PER // EK
NPAD = 10240
SEG = NPAD // NS
NR = 10240
RPT = NR // NS
ZR = 32
B_PER = B // NW
BLK = 2000


def _mesh():
    return plsc.VectorSubcoreMesh(core_axis_name="c", subcore_axis_name="s")


_SC_PARAMS = pltpu.CompilerParams(needs_layout_passes=False)


@functools.partial(
    pl.kernel,
    out_type=jax.ShapeDtypeStruct((NC, NPAD), jnp.float32),
    mesh=_mesh(),
    compiler_params=_SC_PARAMS,
    scratch_types=[
        pltpu.VMEM((NPAD,), jnp.float32),
        pltpu.VMEM((EK,), jnp.int32),
        pltpu.VMEM((SEG,), jnp.float32),
        pltpu.VMEM((SEG,), jnp.float32),
        pltpu.VMEM_SHARED((NS, NPAD), jnp.float32),
    ],
)
def _deg_sc(dst_hbm, deg_hbm, hist, chunk, tmp, accv, shared):
    cid = lax.axis_index("c")
    sid = lax.axis_index("s")
    w = cid * NS + sid
    z16 = jnp.zeros((L,), jnp.float32)
    ones16 = jnp.ones((L,), jnp.float32)

    @pl.loop(0, NPAD // L)
    def _zero(i):
        hist[pl.ds(i * L, L)] = z16

    ebase = w * E_PER

    @pl.loop(0, ECH)
    def _edges(c):
        pltpu.sync_copy(dst_hbm.at[pl.ds(ebase + c * EK, EK)], chunk)
        for j in range(EK // L):
            idx = chunk[pl.ds(j * L, L)]
            plsc.addupdate_scatter(hist, [idx], ones16)

    pltpu.sync_copy(hist, shared.at[sid])
    plsc.subcore_barrier()

    @pl.loop(0, SEG // L)
    def _zacc(i):
        accv[pl.ds(i * L, L)] = z16

    for r in range(NS):
        pltpu.sync_copy(shared.at[r, pl.ds(sid * SEG, SEG)], tmp)

        @pl.loop(0, SEG // L)
        def _add(i):
            accv[pl.ds(i * L, L)] = accv[pl.ds(i * L, L)] + tmp[pl.ds(i * L, L)]

    pltpu.sync_copy(accv, deg_hbm.at[cid, pl.ds(sid * SEG, SEG)])


@functools.partial(
    pl.kernel,
    out_type=jax.ShapeDtypeStruct((NC, NR, D), jnp.float32),
    mesh=_mesh(),
    compiler_params=_SC_PARAMS,
    scratch_types=[
        pltpu.VMEM((EK,), jnp.int32),
        pltpu.VMEM((EK,), jnp.int32),
        pltpu.VMEM((EK, D), jnp.float32),
        pltpu.VMEM((ZR, D), jnp.float32),
        pltpu.VMEM_SHARED((NR, D), jnp.float32),
        pltpu.SemaphoreType.DMA,
    ],
)
def _scatter_sc(u_hbm, src_hbm, dst_hbm, zp_hbm, idx_s, idx_d, rows, zbuf,
                acc, sem):
    cid = lax.axis_index("c")
    sid = lax.axis_index("s")
    w = cid * NS + sid
    z16 = jnp.zeros((L,), jnp.float32)
    for r in range(ZR):
        for cc in range(D // L):
            zbuf[r, pl.ds(cc * L, L)] = z16
    row0 = sid * RPT

    @pl.loop(0, RPT // ZR)
    def _zero(b):
        pltpu.sync_copy(zbuf, acc.at[pl.ds(row0 + b * ZR, ZR)])

    plsc.subcore_barrier()
    ebase = w * E_PER

    @pl.loop(0, ECH)
    def _edges(c):
        pltpu.sync_copy(src_hbm.at[pl.ds(ebase + c * EK, EK)], idx_s)
        pltpu.sync_copy(dst_hbm.at[pl.ds(ebase + c * EK, EK)], idx_d)
        pltpu.async_copy(u_hbm.at[idx_s], rows, sem).wait()
        pltpu.sync_copy(rows, acc.at[idx_d], add=True)

    plsc.subcore_barrier()
    pltpu.sync_copy(acc.at[pl.ds(row0, RPT)],
                    zp_hbm.at[cid, pl.ds(row0, RPT)])


@functools.partial(
    pl.kernel,
    out_type=jax.ShapeDtypeStruct((B, D), jnp.float32),
    mesh=_mesh(),
    compiler_params=_SC_PARAMS,
    scratch_types=[
        pltpu.VMEM((B_PER,), jnp.int32),
        pltpu.VMEM((B_PER, D), jnp.float32),
        pltpu.SemaphoreType.DMA,
    ],
)
def _gather_sc(o_hbm, users_hbm, out_hbm, idx, rows, sem):
    cid = lax.axis_index("c")
    sid = lax.axis_index("s")
    base = (cid * NS + sid) * B_PER
    pltpu.sync_copy(users_hbm.at[pl.ds(base, B_PER)], idx)
    pltpu.async_copy(o_hbm.at[idx], rows, sem).wait()
    pltpu.sync_copy(rows, out_hbm.at[pl.ds(base, B_PER)])


def _tc_layer1(x, W1, dega, degb):
    def body(x_ref, w_ref, da_ref, db_ref, u_ref, dinv_ref):
        dinv = lax.rsqrt(da_ref[...] + db_ref[...] + 1.0)
        dinv_ref[...] = dinv
        u_ref[...] = dinv * jnp.dot(x_ref[...], w_ref[...],
                                    preferred_element_type=jnp.float32)

    return pl.pallas_call(
        body,
        grid=(N // BLK,),
        in_specs=[
            pl.BlockSpec((BLK, D), lambda i: (i, 0)),
            pl.BlockSpec((D, D), lambda i: (0, 0)),
            pl.BlockSpec((BLK, 1), lambda i: (i, 0)),
            pl.BlockSpec((BLK, 1), lambda i: (i, 0)),
        ],
        out_specs=[
            pl.BlockSpec((BLK, D), lambda i: (i, 0)),
            pl.BlockSpec((BLK, 1), lambda i: (i, 0)),
        ],
        out_shape=[
            jax.ShapeDtypeStruct((N, D), jnp.float32),
            jax.ShapeDtypeStruct((N, 1), jnp.float32),
        ],
    )(x, W1, dega, degb)


def _tc_mid(z1, u1, dinv, b1, W2):
    def body(za_ref, zb_ref, u_ref, dinv_ref, b_ref, w_ref, u2_ref):
        dinv = dinv_ref[...]
        h = jnp.tanh(dinv * (za_ref[0] + zb_ref[0] + u_ref[...]) + b_ref[...])
        u2_ref[...] = dinv * jnp.dot(h, w_ref[...],
                                     preferred_element_type=jnp.float32)

    return pl.pallas_call(
        body,
        grid=(N // BLK,),
        in_specs=[
            pl.BlockSpec((1, BLK, D), lambda i: (0, i, 0)),
            pl.BlockSpec((1, BLK, D), lambda i: (1, i, 0)),
            pl.BlockSpec((BLK, D), lambda i: (i, 0)),
            pl.BlockSpec((BLK, 1), lambda i: (i, 0)),
            pl.BlockSpec((1, D), lambda i: (0, 0)),
            pl.BlockSpec((D, D), lambda i: (0, 0)),
        ],
        out_specs=pl.BlockSpec((BLK, D), lambda i: (i, 0)),
        out_shape=jax.ShapeDtypeStruct((N, D), jnp.float32),
    )(z1, z1, u1, dinv, b1, W2)


def _tc_final(z2, u2, dinv, b2):
    def body(za_ref, zb_ref, u_ref, dinv_ref, b_ref, o_ref):
        o_ref[...] = jnp.tanh(
            dinv_ref[...] * (za_ref[0] + zb_ref[0] + u_ref[...]) + b_ref[...])

    return pl.pallas_call(
        body,
        grid=(N // BLK,),
        in_specs=[
            pl.BlockSpec((1, BLK, D), lambda i: (0, i, 0)),
            pl.BlockSpec((1, BLK, D), lambda i: (1, i, 0)),
            pl.BlockSpec((BLK, D), lambda i: (i, 0)),
            pl.BlockSpec((BLK, 1), lambda i: (i, 0)),
            pl.BlockSpec((1, D), lambda i: (0, 0)),
        ],
        out_specs=pl.BlockSpec((BLK, D), lambda i: (i, 0)),
        out_shape=jax.ShapeDtypeStruct((N, D), jnp.float32),
    )(z2, z2, u2, dinv, b2)


def kernel(users, x, edge_index, W1, b1, W2, b2):
    src = edge_index[0]
    dst = edge_index[1]
    degp = _deg_sc(dst)
    dega = degp[0, :N].reshape(N, 1)
    degb = degp[1, :N].reshape(N, 1)
    u1, dinv = _tc_layer1(x, W1, dega, degb)
    z1 = _scatter_sc(u1, src, dst)
    u2 = _tc_mid(z1, u1, dinv, b1.reshape(1, D), W2)
    z2 = _scatter_sc(u2, src, dst)
    o = _tc_final(z2, u2, dinv, b2.reshape(1, D))
    return _gather_sc(o, users)


# speedup vs baseline: 31.4820x; 2.3736x over previous
import functools

import jax
import jax.numpy as jnp
from jax import lax
from jax.experimental import pallas as pl
from jax.experimental.pallas import tpu as pltpu
from jax.experimental.pallas import tpu_sc as plsc

NC = 2
NS = 16
NW = NC * NS
L = 16

N = 10000
E = 320000
D = 128
B = 4096

EK = 128
EPT = 10240
CH = EPT // EK
GP = 8
NG = CH // GP
E2 = NW * EPT
EKD = 80
ECHD = E // NW // EKD
NPAD = 10240
HR = 128
HC = 80
NR = 10240
RPT = NR // NS
B_PER = B // NW
BLK = 2000


def _mesh():
    return plsc.VectorSubcoreMesh(core_axis_name="c", subcore_axis_name="s")


_SC_PARAMS = pltpu.CompilerParams(needs_layout_passes=False)


@functools.partial(
    pl.kernel,
    out_type=jax.ShapeDtypeStruct((NC, HR, HC), jnp.float32),
    mesh=_mesh(),
    compiler_params=_SC_PARAMS,
    scratch_types=[
        pltpu.VMEM((HR, HC), jnp.float32),
        pltpu.VMEM((ECHD, EKD), jnp.int32),
        pltpu.VMEM((HR,), jnp.int32),
        pltpu.VMEM((HR // NS, HC), jnp.float32),
        pltpu.VMEM_SHARED((HR, HC), jnp.float32),
        pltpu.SemaphoreType.DMA,
    ],
)
def _deg_sc(ei_hbm, deg_hbm, hist, dblk, ridx, zrow, shacc, sem):
    cid = lax.axis_index("c")
    sid = lax.axis_index("s")
    w = cid * NS + sid
    pltpu.async_copy(ei_hbm.at[1, w], dblk, sem)
    z16 = jnp.zeros((L,), jnp.float32)
    ones16 = jnp.ones((L,), jnp.float32)
    iota16 = lax.iota(jnp.int32, L)
    for j in range(HR // L):
        ridx[pl.ds(j * L, L)] = iota16 + j * L
    for r in range(HR // NS):
        for j in range(HC // L):
            zrow[r, pl.ds(j * L, L)] = z16

    @pl.loop(0, HR)
    def _zh(r):
        for j in range(HC // L):
            hist[r, pl.ds(j * L, L)] = z16

    pltpu.sync_copy(zrow, shacc.at[pl.ds(sid * (HR // NS), HR // NS)])
    plsc.subcore_barrier()
    pltpu.make_async_copy(ei_hbm.at[1, w], dblk, sem).wait()

    @pl.loop(0, ECHD)
    def _edges(r):
        for j in range(EKD // L):
            idx = dblk[r, pl.ds(j * L, L)]
            qr = idx // HC
            qc = idx - qr * HC
            plsc.addupdate_scatter(hist, [qr, qc], ones16)

    pltpu.sync_copy(hist, shacc.at[ridx], add=True)
    plsc.subcore_barrier()
    pltpu.sync_copy(shacc.at[pl.ds(sid * (HR // NS), HR // NS)],
                    deg_hbm.at[cid, pl.ds(sid * (HR // NS), HR // NS)])


@functools.partial(
    pl.kernel,
    out_type=jax.ShapeDtypeStruct((NC, NR, D), jnp.float32),
    mesh=_mesh(),
    compiler_params=_SC_PARAMS,
    scratch_types=[
        pltpu.VMEM((2 * GP * EK,), jnp.int32),
        pltpu.VMEM((2, GP, EK), jnp.int32),
        pltpu.VMEM((EK, D), jnp.float32),
        pltpu.VMEM((EK, D), jnp.float32),
        pltpu.VMEM_SHARED((NR, D), jnp.float32),
        pltpu.SemaphoreType.DMA,
        pltpu.SemaphoreType.DMA,
        pltpu.SemaphoreType.DMA,
        pltpu.SemaphoreType.DMA,
        pltpu.SemaphoreType.DMA,
        pltpu.SemaphoreType.DMA,
    ],
)
def _scatter_sc(srcp_hbm, dstp_hbm, u_hbm, zp_hbm, sbuf, dbuf, r0, r1,
                acc, sem_i, sem_z, sg0, sg1, ss0, ss1):
    cid = lax.axis_index("c")
    sid = lax.axis_index("s")
    w = cid * NS + sid
    ebase = w * EPT
    rows = (r0, r1)
    sgs = (sg0, sg1)
    sss = (ss0, ss1)
    glen = GP * EK
    pltpu.async_copy(srcp_hbm.at[pl.ds(ebase, glen)],
                     sbuf.at[pl.ds(0, glen)], sem_i)
    pltpu.async_copy(dstp_hbm.at[w, pl.ds(0, GP)], dbuf.at[0], sem_i)
    z16 = jnp.zeros((L,), jnp.float32)

    @pl.loop(0, EK)
    def _zr(r):
        for cc in range(D // L):
            r0[r, pl.ds(cc * L, L)] = z16

    row0 = sid * RPT
    for k in range(RPT // EK):
        pltpu.async_copy(r0, acc.at[pl.ds(row0 + k * EK, EK)], sem_z)
    for k in range(RPT // EK):
        pltpu.make_async_copy(r0, acc.at[pl.ds(row0, EK)], sem_z).wait()
    plsc.subcore_barrier()

    @pl.loop(0, NG)
    def _group(g):
        p = lax.rem(g, 2)
        sbase = p * glen
        pltpu.make_async_copy(srcp_hbm.at[pl.ds(ebase, glen)],
                              sbuf.at[pl.ds(0, glen)], sem_i).wait()
        pltpu.make_async_copy(dstp_hbm.at[w, pl.ds(0, GP)], dbuf.at[0],
                              sem_i).wait()

        @pl.when(g + 1 < NG)
        def _pf():
            off = (g + 1) * glen
            pltpu.async_copy(srcp_hbm.at[pl.ds(ebase + off, glen)],
                             sbuf.at[pl.ds((1 - p) * glen, glen)], sem_i)
            pltpu.async_copy(dstp_hbm.at[w, pl.ds((g + 1) * GP, GP)],
                             dbuf.at[1 - p], sem_i)

        for j in range(GP):
            b = j % 2
            if j < 2:
                @pl.when(g > 0)
                def _free():
                    pltpu.make_async_copy(rows[b], acc.at[dbuf.at[0, 0]],
                                          sss[b]).wait()
            else:
                pltpu.make_async_copy(rows[b], acc.at[dbuf.at[0, 0]],
                                      sss[b]).wait()
            pltpu.async_copy(
                u_hbm.at[sbuf.at[pl.ds(sbase + j * EK, EK)]], rows[b],
                sgs[b])
            if j >= 1:
                bb = (j - 1) % 2
                pltpu.make_async_copy(u_hbm.at[sbuf.at[pl.ds(0, EK)]],
                                      rows[bb], sgs[bb]).wait()
                pltpu.async_copy(rows[bb], acc.at[dbuf.at[p, j - 1]],
                                 sss[bb], add=True)
        bl = (GP - 1) % 2
        pltpu.make_async_copy(u_hbm.at[sbuf.at[pl.ds(0, EK)]], rows[bl],
                              sgs[bl]).wait()
        pltpu.async_copy(rows[bl], acc.at[dbuf.at[p, GP - 1]], sss[bl],
                         add=True)

    for b in range(2):
        pltpu.make_async_copy(rows[b], acc.at[dbuf.at[0, 0]], sss[b]).wait()
    plsc.subcore_barrier()
    pltpu.sync_copy(acc.at[pl.ds(row0, RPT)],
                    zp_hbm.at[cid, pl.ds(row0, RPT)])


@functools.partial(
    pl.kernel,
    out_type=jax.ShapeDtypeStruct((B, D), jnp.float32),
    mesh=_mesh(),
    compiler_params=_SC_PARAMS,
    scratch_types=[
        pltpu.VMEM((B_PER,), jnp.int32),
        pltpu.VMEM((B_PER, D), jnp.float32),
        pltpu.SemaphoreType.DMA,
    ],
)
def _gather_sc(o_hbm, users_hbm, out_hbm, idx, rows, sem):
    cid = lax.axis_index("c")
    sid = lax.axis_index("s")
    base = (cid * NS + sid) * B_PER
    pltpu.sync_copy(users_hbm.at[pl.ds(base, B_PER)], idx)
    pltpu.async_copy(o_hbm.at[idx], rows, sem).wait()
    pltpu.sync_copy(rows, out_hbm.at[pl.ds(base, B_PER)])


def _tc_layer1(x, W1, dega, degb):
    def body(x_ref, w_ref, da_ref, db_ref, u_ref, dinv_ref):
        dinv = lax.rsqrt(da_ref[...] + db_ref[...] + 1.0)
        dinv_ref[...] = dinv
        u_ref[...] = dinv * jnp.dot(x_ref[...], w_ref[...],
                                    preferred_element_type=jnp.float32)

    return pl.pallas_call(
        body,
        grid=(N // BLK,),
        in_specs=[
            pl.BlockSpec((BLK, D), lambda i: (i, 0)),
            pl.BlockSpec((D, D), lambda i: (0, 0)),
            pl.BlockSpec((BLK, 1), lambda i: (i, 0)),
            pl.BlockSpec((BLK, 1), lambda i: (i, 0)),
        ],
        out_specs=[
            pl.BlockSpec((BLK, D), lambda i: (i, 0)),
            pl.BlockSpec((BLK, 1), lambda i: (i, 0)),
        ],
        out_shape=[
            jax.ShapeDtypeStruct((N, D), jnp.float32),
            jax.ShapeDtypeStruct((N, 1), jnp.float32),
        ],
    )(x, W1, dega, degb)


def _tc_mid(z1, u1, dinv, b1, W2):
    def body(za_ref, zb_ref, u_ref, dinv_ref, b_ref, w_ref, u2_ref):
        dinv = dinv_ref[...]
        h = jnp.tanh(dinv * (za_ref[0] + zb_ref[0] + u_ref[...]) + b_ref[...])
        u2_ref[...] = dinv * jnp.dot(h, w_ref[...],
                                     preferred_element_type=jnp.float32)

    return pl.pallas_call(
        body,
        grid=(N // BLK,),
        in_specs=[
            pl.BlockSpec((1, BLK, D), lambda i: (0, i, 0)),
            pl.BlockSpec((1, BLK, D), lambda i: (1, i, 0)),
            pl.BlockSpec((BLK, D), lambda i: (i, 0)),
            pl.BlockSpec((BLK, 1), lambda i: (i, 0)),
            pl.BlockSpec((1, D), lambda i: (0, 0)),
            pl.BlockSpec((D, D), lambda i: (0, 0)),
        ],
        out_specs=pl.BlockSpec((BLK, D), lambda i: (i, 0)),
        out_shape=jax.ShapeDtypeStruct((N, D), jnp.float32),
    )(z1, z1, u1, dinv, b1, W2)


def _tc_final(z2, u2, dinv, b2):
    def body(za_ref, zb_ref, u_ref, dinv_ref, b_ref, o_ref):
        o_ref[...] = jnp.tanh(
            dinv_ref[...] * (za_ref[0] + zb_ref[0] + u_ref[...]) + b_ref[...])

    return pl.pallas_call(
        body,
        grid=(N // BLK,),
        in_specs=[
            pl.BlockSpec((1, BLK, D), lambda i: (0, i, 0)),
            pl.BlockSpec((1, BLK, D), lambda i: (1, i, 0)),
            pl.BlockSpec((BLK, D), lambda i: (i, 0)),
            pl.BlockSpec((BLK, 1), lambda i: (i, 0)),
            pl.BlockSpec((1, D), lambda i: (0, 0)),
        ],
        out_specs=pl.BlockSpec((BLK, D), lambda i: (i, 0)),
        out_shape=jax.ShapeDtypeStruct((N, D), jnp.float32),
    )(z2, z2, u2, dinv, b2)


def _pad_edges(edge_index):
    pad = E2 // NW - E // NW
    tpad = jnp.arange(pad, dtype=jnp.int32)
    wids = jnp.arange(NW, dtype=jnp.int32)[:, None]
    src2 = edge_index[0].reshape(NW, E // NW)
    dst2 = edge_index[1].reshape(NW, E // NW)
    src_pad = (tpad[None, :] + wids * 37) % N
    dst_pad = N + (tpad[None, :] + wids * 7) % (NR - N)
    srcp = jnp.concatenate([src2, src_pad], axis=1).reshape(E2)
    dstp = jnp.concatenate([dst2, dst_pad], axis=1).reshape(NW, CH, EK)
    return srcp, dstp


def kernel(users, x, edge_index, W1, b1, W2, b2):
    ei_deg = edge_index.reshape(2, NW, ECHD, EKD)
    srcp, dstp = _pad_edges(edge_index)
    degp = _deg_sc(ei_deg).reshape(NC, NPAD)
    dega = degp[0, :N].reshape(N, 1)
    degb = degp[1, :N].reshape(N, 1)
    u1, dinv = _tc_layer1(x, W1, dega, degb)
    z1 = _scatter_sc(srcp, dstp, u1)
    u2 = _tc_mid(z1, u1, dinv, b1.reshape(1, D), W2)
    z2 = _scatter_sc(srcp, dstp, u2)
    o = _tc_final(z2, u2, dinv, b2.reshape(1, D))
    return _gather_sc(o, users)


# speedup vs baseline: 32.4468x; 1.0306x over previous
import functools

import jax
import jax.numpy as jnp
from jax import lax
from jax.experimental import pallas as pl
from jax.experimental.pallas import tpu as pltpu
from jax.experimental.pallas import tpu_sc as plsc

NC = 2
NS = 16
NW = NC * NS
L = 16

N = 10000
E = 320000
D = 128
B = 4096

EK = 64
NB = 4
EPT = 10240
CH = EPT // EK
GP = 8
NG = CH // GP
E2 = NW * EPT
EKD = 80
ECHD = E // NW // EKD
NPAD = 10240
HR = 80
HC = 128
NR = 10240
RPT = NR // NS
B_PER = B // NW
BLK = 2000


def _mesh():
    return plsc.VectorSubcoreMesh(core_axis_name="c", subcore_axis_name="s")


_SC_PARAMS = pltpu.CompilerParams(needs_layout_passes=False)


@functools.partial(
    pl.kernel,
    out_type=jax.ShapeDtypeStruct((NC, HR, HC), jnp.float32),
    mesh=_mesh(),
    compiler_params=_SC_PARAMS,
    scratch_types=[
        pltpu.VMEM((HR, HC), jnp.float32),
        pltpu.VMEM((ECHD, EKD), jnp.int32),
        pltpu.VMEM((HR,), jnp.int32),
        pltpu.VMEM((8, HC), jnp.float32),
        pltpu.VMEM_SHARED((HR, HC), jnp.float32),
        pltpu.SemaphoreType.DMA,
    ],
)
def _deg_sc(ei_hbm, deg_hbm, hist, dblk, ridx, zrow, shacc, sem):
    cid = lax.axis_index("c")
    sid = lax.axis_index("s")
    w = cid * NS + sid
    pltpu.async_copy(ei_hbm.at[1, w], dblk, sem)
    z16 = jnp.zeros((L,), jnp.float32)
    ones16 = jnp.ones((L,), jnp.float32)
    iota16 = lax.iota(jnp.int32, L)
    for j in range(HR // L):
        ridx[pl.ds(j * L, L)] = iota16 + j * L
    for r in range(8):
        for j in range(HC // L):
            zrow[r, pl.ds(j * L, L)] = z16

    @pl.loop(0, HR)
    def _zh(r):
        for j in range(HC // L):
            hist[r, pl.ds(j * L, L)] = z16

    @pl.when(sid < 10)
    def _zs():
        pltpu.sync_copy(zrow, shacc.at[pl.ds(sid * 8, 8)])

    plsc.subcore_barrier()
    pltpu.make_async_copy(ei_hbm.at[1, w], dblk, sem).wait()

    @pl.loop(0, ECHD)
    def _edges(r):
        for j in range(EKD // L):
            idx = dblk[r, pl.ds(j * L, L)]
            qr = lax.shift_right_logical(idx, 7)
            qc = lax.bitwise_and(idx, 127)
            plsc.addupdate_scatter(hist, [qr, qc], ones16)

    pltpu.sync_copy(hist, shacc.at[ridx], add=True)
    plsc.subcore_barrier()

    @pl.when(sid < 10)
    def _wb():
        pltpu.sync_copy(shacc.at[pl.ds(sid * 8, 8)],
                        deg_hbm.at[cid, pl.ds(sid * 8, 8)])


@functools.partial(
    pl.kernel,
    out_type=jax.ShapeDtypeStruct((NC, NR, D), jnp.float32),
    mesh=_mesh(),
    compiler_params=_SC_PARAMS,
    scratch_types=[
        pltpu.VMEM((2 * GP * EK,), jnp.int32),
        pltpu.VMEM((2, GP, EK), jnp.int32),
        pltpu.VMEM((EK, D), jnp.float32),
        pltpu.VMEM((EK, D), jnp.float32),
        pltpu.VMEM((EK, D), jnp.float32),
        pltpu.VMEM((EK, D), jnp.float32),
        pltpu.VMEM_SHARED((NR, D), jnp.float32),
        pltpu.SemaphoreType.DMA,
        pltpu.SemaphoreType.DMA,
        pltpu.SemaphoreType.DMA,
        pltpu.SemaphoreType.DMA,
        pltpu.SemaphoreType.DMA,
        pltpu.SemaphoreType.DMA,
        pltpu.SemaphoreType.DMA,
        pltpu.SemaphoreType.DMA,
        pltpu.SemaphoreType.DMA,
        pltpu.SemaphoreType.DMA,
    ],
)
def _scatter_sc(srcp_hbm, dstp_hbm, u_hbm, zp_hbm, sbuf, dbuf,
                r0, r1, r2, r3, acc, sem_i, sem_z,
                sg0, sg1, sg2, sg3, ss0, ss1, ss2, ss3):
    cid = lax.axis_index("c")
    sid = lax.axis_index("s")
    w = cid * NS + sid
    ebase = w * EPT
    rows = (r0, r1, r2, r3)
    sgs = (sg0, sg1, sg2, sg3)
    sss = (ss0, ss1, ss2, ss3)
    glen = GP * EK
    pltpu.async_copy(srcp_hbm.at[pl.ds(ebase, glen)],
                     sbuf.at[pl.ds(0, glen)], sem_i)
    pltpu.async_copy(dstp_hbm.at[w, pl.ds(0, GP)], dbuf.at[0], sem_i)
    z16 = jnp.zeros((L,), jnp.float32)

    @pl.loop(0, EK)
    def _zr(r):
        for cc in range(D // L):
            r0[r, pl.ds(cc * L, L)] = z16

    row0 = sid * RPT
    for k in range(RPT // EK):
        pltpu.async_copy(r0, acc.at[pl.ds(row0 + k * EK, EK)], sem_z)
    for k in range(RPT // EK):
        pltpu.make_async_copy(r0, acc.at[pl.ds(row0, EK)], sem_z).wait()
    plsc.subcore_barrier()

    @pl.loop(0, NG)
    def _group(g):
        p = lax.rem(g, 2)
        sbase = p * glen
        pltpu.make_async_copy(srcp_hbm.at[pl.ds(ebase, glen)],
                              sbuf.at[pl.ds(0, glen)], sem_i).wait()
        pltpu.make_async_copy(dstp_hbm.at[w, pl.ds(0, GP)], dbuf.at[0],
                              sem_i).wait()

        @pl.when(g + 1 < NG)
        def _pf():
            off = (g + 1) * glen
            pltpu.async_copy(srcp_hbm.at[pl.ds(ebase + off, glen)],
                             sbuf.at[pl.ds((1 - p) * glen, glen)], sem_i)
            pltpu.async_copy(dstp_hbm.at[w, pl.ds((g + 1) * GP, GP)],
                             dbuf.at[1 - p], sem_i)

        for j in range(GP):
            b = j % NB
            if j < NB:
                @pl.when(g > 0)
                def _free():
                    pltpu.make_async_copy(rows[b], acc.at[dbuf.at[0, 0]],
                                          sss[b]).wait()
            else:
                pltpu.make_async_copy(rows[b], acc.at[dbuf.at[0, 0]],
                                      sss[b]).wait()
            pltpu.async_copy(
                u_hbm.at[sbuf.at[pl.ds(sbase + j * EK, EK)]], rows[b],
                sgs[b])
            if j >= NB - 1:
                bb = (j - NB + 1) % NB
                pltpu.make_async_copy(u_hbm.at[sbuf.at[pl.ds(0, EK)]],
                                      rows[bb], sgs[bb]).wait()
                pltpu.async_copy(rows[bb], acc.at[dbuf.at[p, j - NB + 1]],
                                 sss[bb], add=True)
        for j in range(GP - NB + 1, GP):
            bl = j % NB
            pltpu.make_async_copy(u_hbm.at[sbuf.at[pl.ds(0, EK)]], rows[bl],
                                  sgs[bl]).wait()
            pltpu.async_copy(rows[bl], acc.at[dbuf.at[p, j]], sss[bl],
                             add=True)

    for b in range(NB):
        pltpu.make_async_copy(rows[b], acc.at[dbuf.at[0, 0]], sss[b]).wait()
    plsc.subcore_barrier()
    pltpu.sync_copy(acc.at[pl.ds(row0, RPT)],
                    zp_hbm.at[cid, pl.ds(row0, RPT)])


@functools.partial(
    pl.kernel,
    out_type=jax.ShapeDtypeStruct((B, D), jnp.float32),
    mesh=_mesh(),
    compiler_params=_SC_PARAMS,
    scratch_types=[
        pltpu.VMEM((B_PER,), jnp.int32),
        pltpu.VMEM((B_PER, D), jnp.float32),
        pltpu.SemaphoreType.DMA,
    ],
)
def _gather_sc(o_hbm, users_hbm, out_hbm, idx, rows, sem):
    cid = lax.axis_index("c")
    sid = lax.axis_index("s")
    base = (cid * NS + sid) * B_PER
    pltpu.sync_copy(users_hbm.at[pl.ds(base, B_PER)], idx)
    pltpu.async_copy(o_hbm.at[idx], rows, sem).wait()
    pltpu.sync_copy(rows, out_hbm.at[pl.ds(base, B_PER)])


def _tc_layer1(x, W1, dega, degb):
    def body(x_ref, w_ref, da_ref, db_ref, u_ref, dinv_ref):
        dinv = lax.rsqrt(da_ref[...] + db_ref[...] + 1.0)
        dinv_ref[...] = dinv
        u_ref[...] = dinv * jnp.dot(x_ref[...], w_ref[...],
                                    preferred_element_type=jnp.float32)

    return pl.pallas_call(
        body,
        grid=(N // BLK,),
        in_specs=[
            pl.BlockSpec((BLK, D), lambda i: (i, 0)),
            pl.BlockSpec((D, D), lambda i: (0, 0)),
            pl.BlockSpec((BLK, 1), lambda i: (i, 0)),
            pl.BlockSpec((BLK, 1), lambda i: (i, 0)),
        ],
        out_specs=[
            pl.BlockSpec((BLK, D), lambda i: (i, 0)),
            pl.BlockSpec((BLK, 1), lambda i: (i, 0)),
        ],
        out_shape=[
            jax.ShapeDtypeStruct((N, D), jnp.float32),
            jax.ShapeDtypeStruct((N, 1), jnp.float32),
        ],
    )(x, W1, dega, degb)


def _tc_mid(z1, u1, dinv, b1, W2):
    def body(za_ref, zb_ref, u_ref, dinv_ref, b_ref, w_ref, u2_ref):
        dinv = dinv_ref[...]
        h = jnp.tanh(dinv * (za_ref[0] + zb_ref[0] + u_ref[...]) + b_ref[...])
        u2_ref[...] = dinv * jnp.dot(h, w_ref[...],
                                     preferred_element_type=jnp.float32)

    return pl.pallas_call(
        body,
        grid=(N // BLK,),
        in_specs=[
            pl.BlockSpec((1, BLK, D), lambda i: (0, i, 0)),
            pl.BlockSpec((1, BLK, D), lambda i: (1, i, 0)),
            pl.BlockSpec((BLK, D), lambda i: (i, 0)),
            pl.BlockSpec((BLK, 1), lambda i: (i, 0)),
            pl.BlockSpec((1, D), lambda i: (0, 0)),
            pl.BlockSpec((D, D), lambda i: (0, 0)),
        ],
        out_specs=pl.BlockSpec((BLK, D), lambda i: (i, 0)),
        out_shape=jax.ShapeDtypeStruct((N, D), jnp.float32),
    )(z1, z1, u1, dinv, b1, W2)


def _tc_final(z2, u2, dinv, b2):
    def body(za_ref, zb_ref, u_ref, dinv_ref, b_ref, o_ref):
        o_ref[...] = jnp.tanh(
            dinv_ref[...] * (za_ref[0] + zb_ref[0] + u_ref[...]) + b_ref[...])

    return pl.pallas_call(
        body,
        grid=(N // BLK,),
        in_specs=[
            pl.BlockSpec((1, BLK, D), lambda i: (0, i, 0)),
            pl.BlockSpec((1, BLK, D), lambda i: (1, i, 0)),
            pl.BlockSpec((BLK, D), lambda i: (i, 0)),
            pl.BlockSpec((BLK, 1), lambda i: (i, 0)),
            pl.BlockSpec((1, D), lambda i: (0, 0)),
        ],
        out_specs=pl.BlockSpec((BLK, D), lambda i: (i, 0)),
        out_shape=jax.ShapeDtypeStruct((N, D), jnp.float32),
    )(z2, z2, u2, dinv, b2)


def _pad_edges(edge_index):
    pad = E2 // NW - E // NW
    tpad = jnp.arange(pad, dtype=jnp.int32)
    wids = jnp.arange(NW, dtype=jnp.int32)[:, None]
    src2 = edge_index[0].reshape(NW, E // NW)
    dst2 = edge_index[1].reshape(NW, E // NW)
    src_pad = (tpad[None, :] + wids * 37) % N
    dst_pad = N + (tpad[None, :] + wids * 7) % (NR - N)
    srcp = jnp.concatenate([src2, src_pad], axis=1).reshape(E2)
    dstp = jnp.concatenate([dst2, dst_pad], axis=1).reshape(NW, CH, EK)
    return srcp, dstp


def kernel(users, x, edge_index, W1, b1, W2, b2):
    ei_deg = edge_index.reshape(2, NW, ECHD, EKD)
    srcp, dstp = _pad_edges(edge_index)
    degp = _deg_sc(ei_deg).reshape(NC, NPAD)
    dega = degp[0, :N].reshape(N, 1)
    degb = degp[1, :N].reshape(N, 1)
    u1, dinv = _tc_layer1(x, W1, dega, degb)
    z1 = _scatter_sc(srcp, dstp, u1)
    u2 = _tc_mid(z1, u1, dinv, b1.reshape(1, D), W2)
    z2 = _scatter_sc(srcp, dstp, u2)
    o = _tc_final(z2, u2, dinv, b2.reshape(1, D))
    return _gather_sc(o, users)


# speedup vs baseline: 34.4712x; 1.0624x over previous
import functools

import jax
import jax.numpy as jnp
from jax import lax
from jax.experimental import pallas as pl
from jax.experimental.pallas import tpu as pltpu
from jax.experimental.pallas import tpu_sc as plsc

NC = 2
NS = 16
NW = NC * NS
L = 16

N = 10000
E = 320000
D = 128
B = 4096

EK = 128
NB = 2
EPT = 10240
CH = EPT // EK
GP = 8
NG = CH // GP
NPAD = 10240
HR = 80
HC = 128
NR = 10240
RPT = NR // NS
B_PER = B // NW
BLK = 2000


def _mesh():
    return plsc.VectorSubcoreMesh(core_axis_name="c", subcore_axis_name="s")


_SC_PARAMS = pltpu.CompilerParams(needs_layout_passes=False)


@functools.partial(
    pl.kernel,
    out_type=jax.ShapeDtypeStruct((NC, HR, HC), jnp.float32),
    mesh=_mesh(),
    compiler_params=_SC_PARAMS,
    scratch_types=[
        pltpu.VMEM((HR, HC), jnp.float32),
        pltpu.VMEM((CH, EK), jnp.int32),
        pltpu.VMEM((HR,), jnp.int32),
        pltpu.VMEM((8, HC), jnp.float32),
        pltpu.VMEM_SHARED((HR, HC), jnp.float32),
        pltpu.SemaphoreType.DMA,
    ],
)
def _deg_sc(ep_hbm, deg_hbm, hist, dblk, ridx, zrow, shacc, sem):
    cid = lax.axis_index("c")
    sid = lax.axis_index("s")
    w = cid * NS + sid
    pltpu.async_copy(ep_hbm.at[1, w], dblk, sem)
    z16 = jnp.zeros((L,), jnp.float32)
    ones16 = jnp.ones((L,), jnp.float32)
    iota16 = lax.iota(jnp.int32, L)
    for j in range(HR // L):
        ridx[pl.ds(j * L, L)] = iota16 + j * L
    for r in range(8):
        for j in range(HC // L):
            zrow[r, pl.ds(j * L, L)] = z16

    @pl.loop(0, HR)
    def _zh(r):
        for j in range(HC // L):
            hist[r, pl.ds(j * L, L)] = z16

    @pl.when(sid < 10)
    def _zs():
        pltpu.sync_copy(zrow, shacc.at[pl.ds(sid * 8, 8)])

    plsc.subcore_barrier()
    pltpu.make_async_copy(ep_hbm.at[1, w], dblk, sem).wait()

    @pl.loop(0, CH)
    def _edges(r):
        for j in range(EK // L):
            idx = dblk[r, pl.ds(j * L, L)]
            qr = lax.shift_right_logical(idx, 7)
            qc = lax.bitwise_and(idx, 127)
            plsc.addupdate_scatter(hist, [qr, qc], ones16)

    pltpu.sync_copy(hist, shacc.at[ridx], add=True)
    plsc.subcore_barrier()

    @pl.when(sid < 10)
    def _wb():
        pltpu.sync_copy(shacc.at[pl.ds(sid * 8, 8)],
                        deg_hbm.at[cid, pl.ds(sid * 8, 8)])


@functools.partial(
    pl.kernel,
    out_type=jax.ShapeDtypeStruct((NC, NR, D), jnp.float32),
    mesh=_mesh(),
    compiler_params=_SC_PARAMS,
    scratch_types=[
        pltpu.VMEM((2, GP, EK), jnp.int32),
        pltpu.VMEM((2, GP, EK), jnp.int32),
        pltpu.VMEM((EK, D), jnp.float32),
        pltpu.VMEM((EK, D), jnp.float32),
        pltpu.VMEM_SHARED((NR, D), jnp.float32),
        pltpu.SemaphoreType.DMA,
        pltpu.SemaphoreType.DMA,
        pltpu.SemaphoreType.DMA,
        pltpu.SemaphoreType.DMA,
        pltpu.SemaphoreType.DMA,
        pltpu.SemaphoreType.DMA,
    ],
)
def _scatter_sc(ep_hbm, u_hbm, zp_hbm, sbuf, dbuf, r0, r1,
                acc, sem_i, sem_z, sg0, sg1, ss0, ss1):
    cid = lax.axis_index("c")
    sid = lax.axis_index("s")
    w = cid * NS + sid
    rows = (r0, r1)
    sgs = (sg0, sg1)
    sss = (ss0, ss1)
    pltpu.async_copy(ep_hbm.at[0, w, pl.ds(0, GP)], sbuf.at[0], sem_i)
    pltpu.async_copy(ep_hbm.at[1, w, pl.ds(0, GP)], dbuf.at[0], sem_i)
    z16 = jnp.zeros((L,), jnp.float32)

    @pl.loop(0, EK)
    def _zr(r):
        for cc in range(D // L):
            r0[r, pl.ds(cc * L, L)] = z16

    row0 = sid * RPT
    for k in range(RPT // EK):
        pltpu.async_copy(r0, acc.at[pl.ds(row0 + k * EK, EK)], sem_z)
    for k in range(RPT // EK):
        pltpu.make_async_copy(r0, acc.at[pl.ds(row0, EK)], sem_z).wait()
    plsc.subcore_barrier()

    @pl.loop(0, NG)
    def _group(g):
        p = lax.rem(g, 2)
        pltpu.make_async_copy(ep_hbm.at[0, w, pl.ds(0, GP)], sbuf.at[0],
                              sem_i).wait()
        pltpu.make_async_copy(ep_hbm.at[1, w, pl.ds(0, GP)], dbuf.at[0],
                              sem_i).wait()

        @pl.when(g + 1 < NG)
        def _pf():
            pltpu.async_copy(ep_hbm.at[0, w, pl.ds((g + 1) * GP, GP)],
                             sbuf.at[1 - p], sem_i)
            pltpu.async_copy(ep_hbm.at[1, w, pl.ds((g + 1) * GP, GP)],
                             dbuf.at[1 - p], sem_i)

        for j in range(GP):
            b = j % NB
            if j < NB:
                @pl.when(g > 0)
                def _free():
                    pltpu.make_async_copy(rows[b], acc.at[dbuf.at[0, 0]],
                                          sss[b]).wait()
            else:
                pltpu.make_async_copy(rows[b], acc.at[dbuf.at[0, 0]],
                                      sss[b]).wait()
            pltpu.async_copy(u_hbm.at[sbuf.at[p, j]], rows[b], sgs[b])
            if j >= NB - 1:
                bb = (j - NB + 1) % NB
                pltpu.make_async_copy(u_hbm.at[sbuf.at[0, 0]],
                                      rows[bb], sgs[bb]).wait()
                pltpu.async_copy(rows[bb], acc.at[dbuf.at[p, j - NB + 1]],
                                 sss[bb], add=True)
        for j in range(GP - NB + 1, GP):
            bl = j % NB
            pltpu.make_async_copy(u_hbm.at[sbuf.at[0, 0]], rows[bl],
                                  sgs[bl]).wait()
            pltpu.async_copy(rows[bl], acc.at[dbuf.at[p, j]], sss[bl],
                             add=True)

    for b in range(NB):
        pltpu.make_async_copy(rows[b], acc.at[dbuf.at[0, 0]], sss[b]).wait()
    plsc.subcore_barrier()
    pltpu.sync_copy(acc.at[pl.ds(row0, RPT)],
                    zp_hbm.at[cid, pl.ds(row0, RPT)])


@functools.partial(
    pl.kernel,
    out_type=jax.ShapeDtypeStruct((B, D), jnp.float32),
    mesh=_mesh(),
    compiler_params=_SC_PARAMS,
    scratch_types=[
        pltpu.VMEM((B_PER,), jnp.int32),
        pltpu.VMEM((B_PER, D), jnp.float32),
        pltpu.SemaphoreType.DMA,
    ],
)
def _gather_sc(o_hbm, users_hbm, out_hbm, idx, rows, sem):
    cid = lax.axis_index("c")
    sid = lax.axis_index("s")
    base = (cid * NS + sid) * B_PER
    pltpu.sync_copy(users_hbm.at[pl.ds(base, B_PER)], idx)
    pltpu.async_copy(o_hbm.at[idx], rows, sem).wait()
    pltpu.sync_copy(rows, out_hbm.at[pl.ds(base, B_PER)])


def _tc_layer1(x, W1, dega, degb):
    def body(x_ref, w_ref, da_ref, db_ref, u_ref, dinv_ref):
        dinv = lax.rsqrt(da_ref[...] + db_ref[...] + 1.0)
        dinv_ref[...] = dinv
        u_ref[...] = dinv * jnp.dot(x_ref[...], w_ref[...],
                                    preferred_element_type=jnp.float32)

    return pl.pallas_call(
        body,
        grid=(N // BLK,),
        in_specs=[
            pl.BlockSpec((BLK, D), lambda i: (i, 0)),
            pl.BlockSpec((D, D), lambda i: (0, 0)),
            pl.BlockSpec((BLK, 1), lambda i: (i, 0)),
            pl.BlockSpec((BLK, 1), lambda i: (i, 0)),
        ],
        out_specs=[
            pl.BlockSpec((BLK, D), lambda i: (i, 0)),
            pl.BlockSpec((BLK, 1), lambda i: (i, 0)),
        ],
        out_shape=[
            jax.ShapeDtypeStruct((N, D), jnp.float32),
            jax.ShapeDtypeStruct((N, 1), jnp.float32),
        ],
    )(x, W1, dega, degb)


def _tc_mid(z1, u1, dinv, b1, W2):
    def body(za_ref, zb_ref, u_ref, dinv_ref, b_ref, w_ref, u2_ref):
        dinv = dinv_ref[...]
        h = jnp.tanh(dinv * (za_ref[0] + zb_ref[0] + u_ref[...]) + b_ref[...])
        u2_ref[...] = dinv * jnp.dot(h, w_ref[...],
                                     preferred_element_type=jnp.float32)

    return pl.pallas_call(
        body,
        grid=(N // BLK,),
        in_specs=[
            pl.BlockSpec((1, BLK, D), lambda i: (0, i, 0)),
            pl.BlockSpec((1, BLK, D), lambda i: (1, i, 0)),
            pl.BlockSpec((BLK, D), lambda i: (i, 0)),
            pl.BlockSpec((BLK, 1), lambda i: (i, 0)),
            pl.BlockSpec((1, D), lambda i: (0, 0)),
            pl.BlockSpec((D, D), lambda i: (0, 0)),
        ],
        out_specs=pl.BlockSpec((BLK, D), lambda i: (i, 0)),
        out_shape=jax.ShapeDtypeStruct((N, D), jnp.float32),
    )(z1, z1, u1, dinv, b1, W2)


def _tc_final(z2, u2, dinv, b2):
    def body(za_ref, zb_ref, u_ref, dinv_ref, b_ref, o_ref):
        o_ref[...] = jnp.tanh(
            dinv_ref[...] * (za_ref[0] + zb_ref[0] + u_ref[...]) + b_ref[...])

    return pl.pallas_call(
        body,
        grid=(N // BLK,),
        in_specs=[
            pl.BlockSpec((1, BLK, D), lambda i: (0, i, 0)),
            pl.BlockSpec((1, BLK, D), lambda i: (1, i, 0)),
            pl.BlockSpec((BLK, D), lambda i: (i, 0)),
            pl.BlockSpec((BLK, 1), lambda i: (i, 0)),
            pl.BlockSpec((1, D), lambda i: (0, 0)),
        ],
        out_specs=pl.BlockSpec((BLK, D), lambda i: (i, 0)),
        out_shape=jax.ShapeDtypeStruct((N, D), jnp.float32),
    )(z2, z2, u2, dinv, b2)


def _pad_edges(edge_index):
    pad = EPT - E // NW
    tpad = jnp.arange(pad, dtype=jnp.int32)[None, None, :]
    wids = jnp.arange(NW, dtype=jnp.int32)[None, :, None]
    kind = jnp.arange(2, dtype=jnp.int32)[:, None, None]
    pads = jnp.where(kind == 0,
                     (tpad + wids * 37) % N,
                     N + (tpad + wids * 7) % (NR - N))
    e2 = edge_index.reshape(2, NW, E // NW)
    return jnp.concatenate([e2, pads], axis=2).reshape(2, NW, CH, EK)


def kernel(users, x, edge_index, W1, b1, W2, b2):
    ep = _pad_edges(edge_index)
    degp = _deg_sc(ep).reshape(NC, NPAD)
    dega = degp[0, :N].reshape(N, 1)
    degb = degp[1, :N].reshape(N, 1)
    u1, dinv = _tc_layer1(x, W1, dega, degb)
    z1 = _scatter_sc(ep, u1)
    u2 = _tc_mid(z1, u1, dinv, b1.reshape(1, D), W2)
    z2 = _scatter_sc(ep, u2)
    o = _tc_final(z2, u2, dinv, b2.reshape(1, D))
    return _gather_sc(o, users)


# speedup vs baseline: 34.9824x; 1.0148x over previous
import functools

import jax
import jax.numpy as jnp
from jax import lax
from jax.experimental import pallas as pl
from jax.experimental.pallas import tpu as pltpu
from jax.experimental.pallas import tpu_sc as plsc

NC = 2
NS = 16
NW = NC * NS
L = 16

N = 10000
E = 320000
D = 128
B = 4096

EK = 128
NB = 2
EPT = 10240
CH = EPT // EK
GP = 8
NG = CH // GP
NPAD = 10240
HR = 80
HC = 128
NR = 10240
RPT = NR // NS
B_PER = B // NW
BLK = 2000


def _mesh():
    return plsc.VectorSubcoreMesh(core_axis_name="c", subcore_axis_name="s")


_SC_PARAMS = pltpu.CompilerParams(needs_layout_passes=False)


@functools.partial(
    pl.kernel,
    out_type=jax.ShapeDtypeStruct((NC, HR, HC), jnp.float32),
    mesh=_mesh(),
    compiler_params=_SC_PARAMS,
    scratch_types=[
        pltpu.VMEM((HR, HC), jnp.float32),
        pltpu.VMEM((CH, EK), jnp.int32),
        pltpu.VMEM((HR,), jnp.int32),
        pltpu.VMEM((8, HC), jnp.float32),
        pltpu.VMEM_SHARED((HR, HC), jnp.float32),
        pltpu.SemaphoreType.DMA,
    ],
)
def _deg_sc(ep_hbm, deg_hbm, hist, dblk, ridx, zrow, shacc, sem):
    cid = lax.axis_index("c")
    sid = lax.axis_index("s")
    w = cid * NS + sid
    pltpu.async_copy(ep_hbm.at[1, w], dblk, sem)
    z16 = jnp.zeros((L,), jnp.float32)
    ones16 = jnp.ones((L,), jnp.float32)
    iota16 = lax.iota(jnp.int32, L)
    for j in range(HR // L):
        ridx[pl.ds(j * L, L)] = iota16 + j * L
    for r in range(8):
        for j in range(HC // L):
            zrow[r, pl.ds(j * L, L)] = z16

    @pl.loop(0, HR)
    def _zh(r):
        for j in range(HC // L):
            hist[r, pl.ds(j * L, L)] = z16

    @pl.when(sid < 10)
    def _zs():
        pltpu.sync_copy(zrow, shacc.at[pl.ds(sid * 8, 8)])

    plsc.subcore_barrier()
    pltpu.make_async_copy(ep_hbm.at[1, w], dblk, sem).wait()

    @pl.loop(0, CH)
    def _edges(r):
        for j in range(EK // L):
            idx = dblk[r, pl.ds(j * L, L)]
            qr = lax.shift_right_logical(idx, 7)
            qc = lax.bitwise_and(idx, 127)
            plsc.addupdate_scatter(hist, [qr, qc], ones16)

    pltpu.sync_copy(hist, shacc.at[ridx], add=True)
    plsc.subcore_barrier()

    @pl.when(sid < 10)
    def _wb():
        pltpu.sync_copy(shacc.at[pl.ds(sid * 8, 8)],
                        deg_hbm.at[cid, pl.ds(sid * 8, 8)])


@functools.partial(
    pl.kernel,
    out_type=jax.ShapeDtypeStruct((NC, NR, D), jnp.float32),
    mesh=_mesh(),
    compiler_params=_SC_PARAMS,
    scratch_types=[
        pltpu.VMEM((2, GP, EK), jnp.int32),
        pltpu.VMEM((2, GP, EK), jnp.int32),
        pltpu.VMEM((EK, D), jnp.float32),
        pltpu.VMEM((EK, D), jnp.float32),
        pltpu.VMEM_SHARED((NR, D), jnp.float32),
        pltpu.SemaphoreType.DMA,
        pltpu.SemaphoreType.DMA,
        pltpu.SemaphoreType.DMA,
        pltpu.SemaphoreType.DMA,
        pltpu.SemaphoreType.DMA,
        pltpu.SemaphoreType.DMA,
    ],
)
def _scatter_sc(ep_hbm, u_hbm, zp_hbm, sbuf, dbuf, r0, r1,
                acc, sem_i, sem_z, sg0, sg1, ss0, ss1):
    cid = lax.axis_index("c")
    sid = lax.axis_index("s")
    w = cid * NS + sid
    rows = (r0, r1)
    sgs = (sg0, sg1)
    sss = (ss0, ss1)
    pltpu.async_copy(ep_hbm.at[0, w, pl.ds(0, GP)], sbuf.at[0], sem_i)
    pltpu.async_copy(ep_hbm.at[1, w, pl.ds(0, GP)], dbuf.at[0], sem_i)
    z16 = jnp.zeros((L,), jnp.float32)

    @pl.loop(0, EK)
    def _zr(r):
        for cc in range(D // L):
            r0[r, pl.ds(cc * L, L)] = z16

    row0 = sid * RPT
    for k in range(RPT // EK):
        pltpu.async_copy(r0, acc.at[pl.ds(row0 + k * EK, EK)], sem_z)
    for k in range(RPT // EK):
        pltpu.make_async_copy(r0, acc.at[pl.ds(row0, EK)], sem_z).wait()
    plsc.subcore_barrier()

    @pl.loop(0, NG)
    def _group(g):
        p = lax.rem(g, 2)
        pltpu.make_async_copy(ep_hbm.at[0, w, pl.ds(0, GP)], sbuf.at[0],
                              sem_i).wait()
        pltpu.make_async_copy(ep_hbm.at[1, w, pl.ds(0, GP)], dbuf.at[0],
                              sem_i).wait()

        @pl.when(g + 1 < NG)
        def _pf():
            pltpu.async_copy(ep_hbm.at[0, w, pl.ds((g + 1) * GP, GP)],
                             sbuf.at[1 - p], sem_i)
            pltpu.async_copy(ep_hbm.at[1, w, pl.ds((g + 1) * GP, GP)],
                             dbuf.at[1 - p], sem_i)

        for j in range(GP):
            b = j % NB
            if j < NB:
                @pl.when(g > 0)
                def _free():
                    pltpu.make_async_copy(rows[b], acc.at[dbuf.at[0, 0]],
                                          sss[b]).wait()
            else:
                pltpu.make_async_copy(rows[b], acc.at[dbuf.at[0, 0]],
                                      sss[b]).wait()
            pltpu.async_copy(u_hbm.at[sbuf.at[p, j]], rows[b], sgs[b])
            if j >= NB - 1:
                bb = (j - NB + 1) % NB
                pltpu.make_async_copy(u_hbm.at[sbuf.at[0, 0]],
                                      rows[bb], sgs[bb]).wait()
                pltpu.async_copy(rows[bb], acc.at[dbuf.at[p, j - NB + 1]],
                                 sss[bb], add=True)
        for j in range(GP - NB + 1, GP):
            bl = j % NB
            pltpu.make_async_copy(u_hbm.at[sbuf.at[0, 0]], rows[bl],
                                  sgs[bl]).wait()
            pltpu.async_copy(rows[bl], acc.at[dbuf.at[p, j]], sss[bl],
                             add=True)

    for b in range(NB):
        pltpu.make_async_copy(rows[b], acc.at[dbuf.at[0, 0]], sss[b]).wait()
    plsc.subcore_barrier()
    pltpu.sync_copy(acc.at[pl.ds(row0, RPT)],
                    zp_hbm.at[cid, pl.ds(row0, RPT)])


@functools.partial(
    pl.kernel,
    out_type=jax.ShapeDtypeStruct((B, D), jnp.float32),
    mesh=_mesh(),
    compiler_params=_SC_PARAMS,
    scratch_types=[
        pltpu.VMEM((B_PER,), jnp.int32),
        pltpu.VMEM((B_PER, D), jnp.float32),
        pltpu.SemaphoreType.DMA,
    ],
)
def _gather_sc(o_hbm, users_hbm, out_hbm, idx, rows, sem):
    cid = lax.axis_index("c")
    sid = lax.axis_index("s")
    base = (cid * NS + sid) * B_PER
    pltpu.sync_copy(users_hbm.at[pl.ds(base, B_PER)], idx)
    pltpu.async_copy(o_hbm.at[idx], rows, sem).wait()
    pltpu.sync_copy(rows, out_hbm.at[pl.ds(base, B_PER)])


def _tc_layer1(x, W1, dcol):
    def body(x_ref, w_ref, dc_ref, u_ref, dinv_ref):
        dinv = lax.rsqrt(dc_ref[...])
        dinv_ref[...] = dinv
        u_ref[...] = dinv * jnp.dot(x_ref[...], w_ref[...],
                                    preferred_element_type=jnp.float32)

    return pl.pallas_call(
        body,
        grid=(N // BLK,),
        in_specs=[
            pl.BlockSpec((BLK, D), lambda i: (i, 0)),
            pl.BlockSpec((D, D), lambda i: (0, 0)),
            pl.BlockSpec((BLK, 1), lambda i: (i, 0)),
        ],
        out_specs=[
            pl.BlockSpec((BLK, D), lambda i: (i, 0)),
            pl.BlockSpec((BLK, 1), lambda i: (i, 0)),
        ],
        out_shape=[
            jax.ShapeDtypeStruct((N, D), jnp.float32),
            jax.ShapeDtypeStruct((N, 1), jnp.float32),
        ],
    )(x, W1, dcol)


def _tc_mid(z1, u1, dinv, b1, W2):
    def body(za_ref, zb_ref, u_ref, dinv_ref, b_ref, w_ref, u2_ref):
        dinv = dinv_ref[...]
        h = jnp.tanh(dinv * (za_ref[0] + zb_ref[0] + u_ref[...]) + b_ref[...])
        u2_ref[...] = dinv * jnp.dot(h, w_ref[...],
                                     preferred_element_type=jnp.float32)

    return pl.pallas_call(
        body,
        grid=(N // BLK,),
        in_specs=[
            pl.BlockSpec((1, BLK, D), lambda i: (0, i, 0)),
            pl.BlockSpec((1, BLK, D), lambda i: (1, i, 0)),
            pl.BlockSpec((BLK, D), lambda i: (i, 0)),
            pl.BlockSpec((BLK, 1), lambda i: (i, 0)),
            pl.BlockSpec((1, D), lambda i: (0, 0)),
            pl.BlockSpec((D, D), lambda i: (0, 0)),
        ],
        out_specs=pl.BlockSpec((BLK, D), lambda i: (i, 0)),
        out_shape=jax.ShapeDtypeStruct((N, D), jnp.float32),
    )(z1, z1, u1, dinv, b1, W2)


def _tc_final(z2, u2, dinv, b2):
    def body(za_ref, zb_ref, u_ref, dinv_ref, b_ref, o_ref):
        o_ref[...] = jnp.tanh(
            dinv_ref[...] * (za_ref[0] + zb_ref[0] + u_ref[...]) + b_ref[...])

    return pl.pallas_call(
        body,
        grid=(N // BLK,),
        in_specs=[
            pl.BlockSpec((1, BLK, D), lambda i: (0, i, 0)),
            pl.BlockSpec((1, BLK, D), lambda i: (1, i, 0)),
            pl.BlockSpec((BLK, D), lambda i: (i, 0)),
            pl.BlockSpec((BLK, 1), lambda i: (i, 0)),
            pl.BlockSpec((1, D), lambda i: (0, 0)),
        ],
        out_specs=pl.BlockSpec((BLK, D), lambda i: (i, 0)),
        out_shape=jax.ShapeDtypeStruct((N, D), jnp.float32),
    )(z2, z2, u2, dinv, b2)


def _pad_edges(edge_index):
    pad = EPT - E // NW
    tpad = jnp.arange(pad, dtype=jnp.int32)[None, None, :]
    wids = jnp.arange(NW, dtype=jnp.int32)[None, :, None]
    kind = jnp.arange(2, dtype=jnp.int32)[:, None, None]
    pads = jnp.where(kind == 0,
                     (tpad + wids * 37) % N,
                     N + (tpad + wids * 7) % (NR - N))
    e2 = edge_index.reshape(2, NW, E // NW)
    return jnp.concatenate([e2, pads], axis=2).reshape(2, NW, CH, EK)


def kernel(users, x, edge_index, W1, b1, W2, b2):
    ep = _pad_edges(edge_index)
    degp = _deg_sc(ep).reshape(NC, NPAD)
    dcol = (degp[0, :N] + degp[1, :N] + 1.0).reshape(N, 1)
    u1, dinv = _tc_layer1(x, W1, dcol)
    z1 = _scatter_sc(ep, u1)
    u2 = _tc_mid(z1, u1, dinv, b1.reshape(1, D), W2)
    z2 = _scatter_sc(ep, u2)
    o = _tc_final(z2, u2, dinv, b2.reshape(1, D))
    return _gather_sc(o, users)


# speedup vs baseline: 35.4876x; 1.0144x over previous
import functools

import jax
import jax.numpy as jnp
from jax import lax
from jax.experimental import pallas as pl
from jax.experimental.pallas import tpu as pltpu
from jax.experimental.pallas import tpu_sc as plsc

NC = 2
NS = 16
NW = NC * NS
L = 16

N = 10000
E = 320000
D = 128
B = 4096

EK = 128
NB = 2
EPT = 10240
CH = EPT // EK
GP = 8
NG = CH // GP
NPAD = 10240
HR = 80
HC = 128
NR = 10240
RPT = NR // NS
B_PER = B // NW
BLK = 2000


def _mesh():
    return plsc.VectorSubcoreMesh(core_axis_name="c", subcore_axis_name="s")


_SC_PARAMS = pltpu.CompilerParams(needs_layout_passes=False)


@functools.partial(
    pl.kernel,
    out_type=jax.ShapeDtypeStruct((NC, HR, HC), jnp.float32),
    mesh=_mesh(),
    compiler_params=_SC_PARAMS,
    scratch_types=[
        pltpu.VMEM((HR, HC), jnp.float32),
        pltpu.VMEM((CH, EK), jnp.int32),
        pltpu.VMEM((HR,), jnp.int32),
        pltpu.VMEM((8, HC), jnp.float32),
        pltpu.VMEM_SHARED((HR, HC), jnp.float32),
        pltpu.SemaphoreType.DMA,
    ],
)
def _deg_sc(ep_hbm, deg_hbm, hist, dblk, ridx, zrow, shacc, sem):
    cid = lax.axis_index("c")
    sid = lax.axis_index("s")
    w = cid * NS + sid
    pltpu.async_copy(ep_hbm.at[1, w], dblk, sem)
    z16 = jnp.zeros((L,), jnp.float32)
    ones16 = jnp.ones((L,), jnp.float32)
    iota16 = lax.iota(jnp.int32, L)
    for j in range(HR // L):
        ridx[pl.ds(j * L, L)] = iota16 + j * L
    for r in range(8):
        for j in range(HC // L):
            zrow[r, pl.ds(j * L, L)] = z16

    @pl.loop(0, HR)
    def _zh(r):
        for j in range(HC // L):
            hist[r, pl.ds(j * L, L)] = z16

    @pl.when(sid < 10)
    def _zs():
        pltpu.sync_copy(zrow, shacc.at[pl.ds(sid * 8, 8)])

    plsc.subcore_barrier()
    pltpu.make_async_copy(ep_hbm.at[1, w], dblk, sem).wait()

    @pl.loop(0, CH)
    def _edges(r):
        for j in range(EK // L):
            idx = dblk[r, pl.ds(j * L, L)]
            qr = lax.shift_right_logical(idx, 7)
            qc = lax.bitwise_and(idx, 127)
            plsc.addupdate_scatter(hist, [qr, qc], ones16)

    pltpu.sync_copy(hist, shacc.at[ridx], add=True)
    plsc.subcore_barrier()

    @pl.when(sid < 10)
    def _wb():
        pltpu.sync_copy(shacc.at[pl.ds(sid * 8, 8)],
                        deg_hbm.at[cid, pl.ds(sid * 8, 8)])


@functools.partial(
    pl.kernel,
    out_type=jax.ShapeDtypeStruct((NC, NR, D), jnp.float32),
    mesh=_mesh(),
    compiler_params=_SC_PARAMS,
    scratch_types=[
        pltpu.VMEM((2, GP, EK), jnp.int32),
        pltpu.VMEM((2, GP, EK), jnp.int32),
        pltpu.VMEM((EK, D), jnp.float32),
        pltpu.VMEM((EK, D), jnp.float32),
        pltpu.VMEM_SHARED((NR, D), jnp.float32),
        pltpu.SemaphoreType.DMA,
        pltpu.SemaphoreType.DMA,
        pltpu.SemaphoreType.DMA,
        pltpu.SemaphoreType.DMA,
        pltpu.SemaphoreType.DMA,
        pltpu.SemaphoreType.DMA,
    ],
)
def _scatter_sc(ep_hbm, u_hbm, zp_hbm, sbuf, dbuf, r0, r1,
                acc, sem_i, sem_z, sg0, sg1, ss0, ss1):
    cid = lax.axis_index("c")
    sid = lax.axis_index("s")
    w = cid * NS + sid
    rows = (r0, r1)
    sgs = (sg0, sg1)
    sss = (ss0, ss1)
    pltpu.async_copy(ep_hbm.at[0, w, pl.ds(0, GP)], sbuf.at[0], sem_i)
    pltpu.async_copy(ep_hbm.at[1, w, pl.ds(0, GP)], dbuf.at[0], sem_i)
    z16 = jnp.zeros((L,), jnp.float32)

    @pl.loop(0, EK)
    def _zr(r):
        for cc in range(D // L):
            r1[r, pl.ds(cc * L, L)] = z16

    row0 = sid * RPT
    for k in range(RPT // EK):
        pltpu.async_copy(r1, acc.at[pl.ds(row0 + k * EK, EK)], sem_z)
    pltpu.make_async_copy(ep_hbm.at[0, w, pl.ds(0, GP)], sbuf.at[0],
                          sem_i).wait()
    pltpu.make_async_copy(ep_hbm.at[1, w, pl.ds(0, GP)], dbuf.at[0],
                          sem_i).wait()
    pltpu.async_copy(ep_hbm.at[0, w, pl.ds(GP, GP)], sbuf.at[1], sem_i)
    pltpu.async_copy(ep_hbm.at[1, w, pl.ds(GP, GP)], dbuf.at[1], sem_i)
    pltpu.async_copy(u_hbm.at[sbuf.at[0, 0]], r0, sg0)
    for k in range(RPT // EK):
        pltpu.make_async_copy(r1, acc.at[pl.ds(row0, EK)], sem_z).wait()
    pltpu.async_copy(u_hbm.at[sbuf.at[0, 1]], r1, sg1)
    plsc.subcore_barrier()

    @pl.loop(0, NG)
    def _group(g):
        p = lax.rem(g, 2)

        @pl.when(g > 0)
        def _ipf():
            pltpu.make_async_copy(ep_hbm.at[0, w, pl.ds(0, GP)], sbuf.at[0],
                                  sem_i).wait()
            pltpu.make_async_copy(ep_hbm.at[1, w, pl.ds(0, GP)], dbuf.at[0],
                                  sem_i).wait()

            @pl.when(g + 1 < NG)
            def _pf():
                pltpu.async_copy(ep_hbm.at[0, w, pl.ds((g + 1) * GP, GP)],
                                 sbuf.at[1 - p], sem_i)
                pltpu.async_copy(ep_hbm.at[1, w, pl.ds((g + 1) * GP, GP)],
                                 dbuf.at[1 - p], sem_i)

        for j in range(GP):
            b = j % NB
            if j < NB:
                @pl.when(g > 0)
                def _free():
                    pltpu.make_async_copy(rows[b], acc.at[dbuf.at[0, 0]],
                                          sss[b]).wait()
                    pltpu.async_copy(u_hbm.at[sbuf.at[p, j]], rows[b],
                                     sgs[b])
            else:
                pltpu.make_async_copy(rows[b], acc.at[dbuf.at[0, 0]],
                                      sss[b]).wait()
                pltpu.async_copy(u_hbm.at[sbuf.at[p, j]], rows[b], sgs[b])
            if j >= NB - 1:
                bb = (j - NB + 1) % NB
                pltpu.make_async_copy(u_hbm.at[sbuf.at[0, 0]],
                                      rows[bb], sgs[bb]).wait()
                pltpu.async_copy(rows[bb], acc.at[dbuf.at[p, j - NB + 1]],
                                 sss[bb], add=True)
        for j in range(GP - NB + 1, GP):
            bl = j % NB
            pltpu.make_async_copy(u_hbm.at[sbuf.at[0, 0]], rows[bl],
                                  sgs[bl]).wait()
            pltpu.async_copy(rows[bl], acc.at[dbuf.at[p, j]], sss[bl],
                             add=True)

    for b in range(NB):
        pltpu.make_async_copy(rows[b], acc.at[dbuf.at[0, 0]], sss[b]).wait()
    plsc.subcore_barrier()
    pltpu.sync_copy(acc.at[pl.ds(row0, RPT)],
                    zp_hbm.at[cid, pl.ds(row0, RPT)])


@functools.partial(
    pl.kernel,
    out_type=jax.ShapeDtypeStruct((B, D), jnp.float32),
    mesh=_mesh(),
    compiler_params=_SC_PARAMS,
    scratch_types=[
        pltpu.VMEM((B_PER,), jnp.int32),
        pltpu.VMEM((B_PER, D), jnp.float32),
        pltpu.SemaphoreType.DMA,
    ],
)
def _gather_sc(o_hbm, users_hbm, out_hbm, idx, rows, sem):
    cid = lax.axis_index("c")
    sid = lax.axis_index("s")
    base = (cid * NS + sid) * B_PER
    pltpu.sync_copy(users_hbm.at[pl.ds(base, B_PER)], idx)
    pltpu.async_copy(o_hbm.at[idx], rows, sem).wait()
    pltpu.sync_copy(rows, out_hbm.at[pl.ds(base, B_PER)])


def _tc_layer1(x, W1, dcol):
    def body(x_ref, w_ref, dc_ref, u_ref):
        dinv = lax.rsqrt(dc_ref[...])
        u_ref[...] = dinv * jnp.dot(x_ref[...], w_ref[...],
                                    preferred_element_type=jnp.float32)

    return pl.pallas_call(
        body,
        grid=(N // BLK,),
        in_specs=[
            pl.BlockSpec((BLK, D), lambda i: (i, 0)),
            pl.BlockSpec((D, D), lambda i: (0, 0)),
            pl.BlockSpec((BLK, 1), lambda i: (i, 0)),
        ],
        out_specs=pl.BlockSpec((BLK, D), lambda i: (i, 0)),
        out_shape=jax.ShapeDtypeStruct((N, D), jnp.float32),
    )(x, W1, dcol)


def _tc_mid(z1, u1, dcol, b1, W2):
    def body(za_ref, zb_ref, u_ref, dc_ref, b_ref, w_ref, u2_ref):
        dinv = lax.rsqrt(dc_ref[...])
        h = jnp.tanh(dinv * (za_ref[0] + zb_ref[0] + u_ref[...]) + b_ref[...])
        u2_ref[...] = dinv * jnp.dot(h, w_ref[...],
                                     preferred_element_type=jnp.float32)

    return pl.pallas_call(
        body,
        grid=(N // BLK,),
        in_specs=[
            pl.BlockSpec((1, BLK, D), lambda i: (0, i, 0)),
            pl.BlockSpec((1, BLK, D), lambda i: (1, i, 0)),
            pl.BlockSpec((BLK, D), lambda i: (i, 0)),
            pl.BlockSpec((BLK, 1), lambda i: (i, 0)),
            pl.BlockSpec((1, D), lambda i: (0, 0)),
            pl.BlockSpec((D, D), lambda i: (0, 0)),
        ],
        out_specs=pl.BlockSpec((BLK, D), lambda i: (i, 0)),
        out_shape=jax.ShapeDtypeStruct((N, D), jnp.float32),
    )(z1, z1, u1, dcol, b1, W2)


def _tc_final(z2, u2, dcol, b2):
    def body(za_ref, zb_ref, u_ref, dc_ref, b_ref, o_ref):
        o_ref[...] = jnp.tanh(
            lax.rsqrt(dc_ref[...]) * (za_ref[0] + zb_ref[0] + u_ref[...])
            + b_ref[...])

    return pl.pallas_call(
        body,
        grid=(N // BLK,),
        in_specs=[
            pl.BlockSpec((1, BLK, D), lambda i: (0, i, 0)),
            pl.BlockSpec((1, BLK, D), lambda i: (1, i, 0)),
            pl.BlockSpec((BLK, D), lambda i: (i, 0)),
            pl.BlockSpec((BLK, 1), lambda i: (i, 0)),
            pl.BlockSpec((1, D), lambda i: (0, 0)),
        ],
        out_specs=pl.BlockSpec((BLK, D), lambda i: (i, 0)),
        out_shape=jax.ShapeDtypeStruct((N, D), jnp.float32),
    )(z2, z2, u2, dcol, b2)


def _pad_edges(edge_index):
    pad = EPT - E // NW
    tpad = jnp.arange(pad, dtype=jnp.int32)[None, None, :]
    wids = jnp.arange(NW, dtype=jnp.int32)[None, :, None]
    kind = jnp.arange(2, dtype=jnp.int32)[:, None, None]
    pads = jnp.where(kind == 0,
                     (tpad + wids * 37) % N,
                     N + (tpad + wids * 7) % (NR - N))
    e2 = edge_index.reshape(2, NW, E // NW)
    return jnp.concatenate([e2, pads], axis=2).reshape(2, NW, CH, EK)


def kernel(users, x, edge_index, W1, b1, W2, b2):
    ep = _pad_edges(edge_index)
    degp = _deg_sc(ep).reshape(NC, NPAD)
    dcol = (degp[0, :N] + degp[1, :N] + 1.0).reshape(N, 1)
    u1 = _tc_layer1(x, W1, dcol)
    z1 = _scatter_sc(ep, u1)
    u2 = _tc_mid(z1, u1, dcol, b1.reshape(1, D), W2)
    z2 = _scatter_sc(ep, u2)
    o = _tc_final(z2, u2, dcol, b2.reshape(1, D))
    return _gather_sc(o, users)


# speedup vs baseline: 35.9274x; 1.0124x over previous
import functools

import jax
import jax.numpy as jnp
from jax import lax
from jax.experimental import pallas as pl
from jax.experimental.pallas import tpu as pltpu
from jax.experimental.pallas import tpu_sc as plsc

NC = 2
NS = 16
NW = NC * NS
L = 16

N = 10000
E = 320000
D = 128
B = 4096

EK = 128
NB = 2
EPT = 10240
CH = EPT // EK
GP = 16
NG = CH // GP
NPAD = 10240
HR = 80
HC = 128
NR = 10240
RPT = NR // NS
B_PER = B // NW
BLK = 2000


def _mesh():
    return plsc.VectorSubcoreMesh(core_axis_name="c", subcore_axis_name="s")


_SC_PARAMS = pltpu.CompilerParams(needs_layout_passes=False)


@functools.partial(
    pl.kernel,
    out_type=jax.ShapeDtypeStruct((NC, HR, HC), jnp.float32),
    mesh=_mesh(),
    compiler_params=_SC_PARAMS,
    scratch_types=[
        pltpu.VMEM((HR, HC), jnp.float32),
        pltpu.VMEM((CH, EK), jnp.int32),
        pltpu.VMEM((HR,), jnp.int32),
        pltpu.VMEM((8, HC), jnp.float32),
        pltpu.VMEM_SHARED((HR, HC), jnp.float32),
        pltpu.SemaphoreType.DMA,
    ],
)
def _deg_sc(ep_hbm, deg_hbm, hist, dblk, ridx, zrow, shacc, sem):
    cid = lax.axis_index("c")
    sid = lax.axis_index("s")
    w = cid * NS + sid
    pltpu.async_copy(ep_hbm.at[1, w], dblk, sem)
    z16 = jnp.zeros((L,), jnp.float32)
    ones16 = jnp.ones((L,), jnp.float32)
    iota16 = lax.iota(jnp.int32, L)
    for j in range(HR // L):
        ridx[pl.ds(j * L, L)] = iota16 + j * L
    for r in range(8):
        for j in range(HC // L):
            zrow[r, pl.ds(j * L, L)] = z16

    @pl.loop(0, HR)
    def _zh(r):
        for j in range(HC // L):
            hist[r, pl.ds(j * L, L)] = z16

    @pl.when(sid < 10)
    def _zs():
        pltpu.sync_copy(zrow, shacc.at[pl.ds(sid * 8, 8)])

    plsc.subcore_barrier()
    pltpu.make_async_copy(ep_hbm.at[1, w], dblk, sem).wait()

    @pl.loop(0, CH)
    def _edges(r):
        for j in range(EK // L):
            idx = dblk[r, pl.ds(j * L, L)]
            qr = lax.shift_right_logical(idx, 7)
            qc = lax.bitwise_and(idx, 127)
            plsc.addupdate_scatter(hist, [qr, qc], ones16)

    pltpu.sync_copy(hist, shacc.at[ridx], add=True)
    plsc.subcore_barrier()

    @pl.when(sid < 10)
    def _wb():
        pltpu.sync_copy(shacc.at[pl.ds(sid * 8, 8)],
                        deg_hbm.at[cid, pl.ds(sid * 8, 8)])


@functools.partial(
    pl.kernel,
    out_type=jax.ShapeDtypeStruct((NC, NR, D), jnp.float32),
    mesh=_mesh(),
    compiler_params=_SC_PARAMS,
    scratch_types=[
        pltpu.VMEM((2, GP, EK), jnp.int32),
        pltpu.VMEM((2, GP, EK), jnp.int32),
        pltpu.VMEM((EK, D), jnp.float32),
        pltpu.VMEM((EK, D), jnp.float32),
        pltpu.VMEM_SHARED((NR, D), jnp.float32),
        pltpu.SemaphoreType.DMA,
        pltpu.SemaphoreType.DMA,
        pltpu.SemaphoreType.DMA,
        pltpu.SemaphoreType.DMA,
        pltpu.SemaphoreType.DMA,
        pltpu.SemaphoreType.DMA,
    ],
)
def _scatter_sc(ep_hbm, u_hbm, zp_hbm, sbuf, dbuf, r0, r1,
                acc, sem_i, sem_z, sg0, sg1, ss0, ss1):
    cid = lax.axis_index("c")
    sid = lax.axis_index("s")
    w = cid * NS + sid
    rows = (r0, r1)
    sgs = (sg0, sg1)
    sss = (ss0, ss1)
    pltpu.async_copy(ep_hbm.at[0, w, pl.ds(0, GP)], sbuf.at[0], sem_i)
    pltpu.async_copy(ep_hbm.at[1, w, pl.ds(0, GP)], dbuf.at[0], sem_i)
    z16 = jnp.zeros((L,), jnp.float32)

    @pl.loop(0, EK)
    def _zr(r):
        for cc in range(D // L):
            r1[r, pl.ds(cc * L, L)] = z16

    row0 = sid * RPT
    for k in range(RPT // EK):
        pltpu.async_copy(r1, acc.at[pl.ds(row0 + k * EK, EK)], sem_z)
    pltpu.make_async_copy(ep_hbm.at[0, w, pl.ds(0, GP)], sbuf.at[0],
                          sem_i).wait()
    pltpu.make_async_copy(ep_hbm.at[1, w, pl.ds(0, GP)], dbuf.at[0],
                          sem_i).wait()
    pltpu.async_copy(ep_hbm.at[0, w, pl.ds(GP, GP)], sbuf.at[1], sem_i)
    pltpu.async_copy(ep_hbm.at[1, w, pl.ds(GP, GP)], dbuf.at[1], sem_i)
    pltpu.async_copy(u_hbm.at[sbuf.at[0, 0]], r0, sg0)
    for k in range(RPT // EK):
        pltpu.make_async_copy(r1, acc.at[pl.ds(row0, EK)], sem_z).wait()
    pltpu.async_copy(u_hbm.at[sbuf.at[0, 1]], r1, sg1)
    plsc.subcore_barrier()

    @pl.loop(0, NG)
    def _group(g):
        p = lax.rem(g, 2)

        @pl.when(g > 0)
        def _ipf():
            pltpu.make_async_copy(ep_hbm.at[0, w, pl.ds(0, GP)], sbuf.at[0],
                                  sem_i).wait()
            pltpu.make_async_copy(ep_hbm.at[1, w, pl.ds(0, GP)], dbuf.at[0],
                                  sem_i).wait()

            @pl.when(g + 1 < NG)
            def _pf():
                pltpu.async_copy(ep_hbm.at[0, w, pl.ds((g + 1) * GP, GP)],
                                 sbuf.at[1 - p], sem_i)
                pltpu.async_copy(ep_hbm.at[1, w, pl.ds((g + 1) * GP, GP)],
                                 dbuf.at[1 - p], sem_i)

        for j in range(GP):
            b = j % NB
            if j < NB:
                @pl.when(g > 0)
                def _free():
                    pltpu.make_async_copy(rows[b], acc.at[dbuf.at[0, 0]],
                                          sss[b]).wait()
                    pltpu.async_copy(u_hbm.at[sbuf.at[p, j]], rows[b],
                                     sgs[b])
            else:
                pltpu.make_async_copy(rows[b], acc.at[dbuf.at[0, 0]],
                                      sss[b]).wait()
                pltpu.async_copy(u_hbm.at[sbuf.at[p, j]], rows[b], sgs[b])
            if j >= NB - 1:
                bb = (j - NB + 1) % NB
                pltpu.make_async_copy(u_hbm.at[sbuf.at[0, 0]],
                                      rows[bb], sgs[bb]).wait()
                pltpu.async_copy(rows[bb], acc.at[dbuf.at[p, j - NB + 1]],
                                 sss[bb], add=True)
        for j in range(GP - NB + 1, GP):
            bl = j % NB
            pltpu.make_async_copy(u_hbm.at[sbuf.at[0, 0]], rows[bl],
                                  sgs[bl]).wait()
            pltpu.async_copy(rows[bl], acc.at[dbuf.at[p, j]], sss[bl],
                             add=True)

    for b in range(NB):
        pltpu.make_async_copy(rows[b], acc.at[dbuf.at[0, 0]], sss[b]).wait()
    plsc.subcore_barrier()
    pltpu.sync_copy(acc.at[pl.ds(row0, RPT)],
                    zp_hbm.at[cid, pl.ds(row0, RPT)])


@functools.partial(
    pl.kernel,
    out_type=jax.ShapeDtypeStruct((B, D), jnp.float32),
    mesh=_mesh(),
    compiler_params=_SC_PARAMS,
    scratch_types=[
        pltpu.VMEM((B_PER,), jnp.int32),
        pltpu.VMEM((B_PER, D), jnp.float32),
        pltpu.SemaphoreType.DMA,
    ],
)
def _gather_sc(o_hbm, users_hbm, out_hbm, idx, rows, sem):
    cid = lax.axis_index("c")
    sid = lax.axis_index("s")
    base = (cid * NS + sid) * B_PER
    pltpu.sync_copy(users_hbm.at[pl.ds(base, B_PER)], idx)
    pltpu.async_copy(o_hbm.at[idx], rows, sem).wait()
    pltpu.sync_copy(rows, out_hbm.at[pl.ds(base, B_PER)])


def _tc_layer1(x, W1, dcol):
    def body(x_ref, w_ref, dc_ref, u_ref):
        dinv = lax.rsqrt(dc_ref[...])
        u_ref[...] = dinv * jnp.dot(x_ref[...], w_ref[...],
                                    preferred_element_type=jnp.float32)

    return pl.pallas_call(
        body,
        grid=(N // BLK,),
        in_specs=[
            pl.BlockSpec((BLK, D), lambda i: (i, 0)),
            pl.BlockSpec((D, D), lambda i: (0, 0)),
            pl.BlockSpec((BLK, 1), lambda i: (i, 0)),
        ],
        out_specs=pl.BlockSpec((BLK, D), lambda i: (i, 0)),
        out_shape=jax.ShapeDtypeStruct((N, D), jnp.float32),
    )(x, W1, dcol)


def _tc_mid(z1, u1, dcol, b1, W2):
    def body(za_ref, zb_ref, u_ref, dc_ref, b_ref, w_ref, u2_ref):
        dinv = lax.rsqrt(dc_ref[...])
        h = jnp.tanh(dinv * (za_ref[0] + zb_ref[0] + u_ref[...]) + b_ref[...])
        u2_ref[...] = dinv * jnp.dot(h, w_ref[...],
                                     preferred_element_type=jnp.float32)

    return pl.pallas_call(
        body,
        grid=(N // BLK,),
        in_specs=[
            pl.BlockSpec((1, BLK, D), lambda i: (0, i, 0)),
            pl.BlockSpec((1, BLK, D), lambda i: (1, i, 0)),
            pl.BlockSpec((BLK, D), lambda i: (i, 0)),
            pl.BlockSpec((BLK, 1), lambda i: (i, 0)),
            pl.BlockSpec((1, D), lambda i: (0, 0)),
            pl.BlockSpec((D, D), lambda i: (0, 0)),
        ],
        out_specs=pl.BlockSpec((BLK, D), lambda i: (i, 0)),
        out_shape=jax.ShapeDtypeStruct((N, D), jnp.float32),
    )(z1, z1, u1, dcol, b1, W2)


def _tc_final(z2, u2, dcol, b2):
    def body(za_ref, zb_ref, u_ref, dc_ref, b_ref, o_ref):
        o_ref[...] = jnp.tanh(
            lax.rsqrt(dc_ref[...]) * (za_ref[0] + zb_ref[0] + u_ref[...])
            + b_ref[...])

    return pl.pallas_call(
        body,
        grid=(N // BLK,),
        in_specs=[
            pl.BlockSpec((1, BLK, D), lambda i: (0, i, 0)),
            pl.BlockSpec((1, BLK, D), lambda i: (1, i, 0)),
            pl.BlockSpec((BLK, D), lambda i: (i, 0)),
            pl.BlockSpec((BLK, 1), lambda i: (i, 0)),
            pl.BlockSpec((1, D), lambda i: (0, 0)),
        ],
        out_specs=pl.BlockSpec((BLK, D), lambda i: (i, 0)),
        out_shape=jax.ShapeDtypeStruct((N, D), jnp.float32),
    )(z2, z2, u2, dcol, b2)


def _pad_edges(edge_index):
    pad = EPT - E // NW
    tpad = jnp.arange(pad, dtype=jnp.int32)[None, None, :]
    wids = jnp.arange(NW, dtype=jnp.int32)[None, :, None]
    kind = jnp.arange(2, dtype=jnp.int32)[:, None, None]
    pads = jnp.where(kind == 0,
                     (tpad + wids * 37) % N,
                     N + (tpad + wids * 7) % (NR - N))
    e2 = edge_index.reshape(2, NW, E // NW)
    return jnp.concatenate([e2, pads], axis=2).reshape(2, NW, CH, EK)


def kernel(users, x, edge_index, W1, b1, W2, b2):
    ep = _pad_edges(edge_index)
    degp = _deg_sc(ep).reshape(NC, NPAD)
    dcol = (degp[0, :N] + degp[1, :N] + 1.0).reshape(N, 1)
    u1 = _tc_layer1(x, W1, dcol)
    z1 = _scatter_sc(ep, u1)
    u2 = _tc_mid(z1, u1, dcol, b1.reshape(1, D), W2)
    z2 = _scatter_sc(ep, u2)
    o = _tc_final(z2, u2, dcol, b2.reshape(1, D))
    return _gather_sc(o, users)
